# Initial kernel scaffold; baseline (speedup 1.0000x reference)
import jax
import jax.numpy as jnp
from jax.experimental import pallas as pl


def kernel(node_feat, edge_attr, edge_index, batch, Wn1, bn1, Wn2, bn2, We1, be1, We2, be2, wq, Wu1, bu1, Wu2, bu2, Wh1, bh1, Wh2, bh2):
    raise NotImplementedError("write your pallas kernel here")


# speedup vs baseline: 584.4290x; 584.4290x over previous
import functools
import math

import jax
import jax.numpy as jnp
from jax import lax
from jax.experimental import pallas as pl
from jax.experimental.pallas import tpu as pltpu
from jax.experimental.pallas import tpu_sc as plsc

PI = math.pi

_N_NODES = 10000
_N_PAD = 10240
_N_SENT = 10256
_N_EDGES = 160000
_N_OCC = 320000
_K = 3
_NG = 64

_SC_PARAMS = pltpu.CompilerParams(needs_layout_passes=False)
_F32 = jnp.float32
_I32 = jnp.int32


def _leaky(x):
    return jnp.where(x >= 0, x, 0.01 * x)


def _node_mlp_body(x_ref, w1_ref, b1_ref, w2_ref, b2_ref, o_ref):
    h = _leaky(jnp.dot(x_ref[...], w1_ref[...], preferred_element_type=_F32)
               + b1_ref[...])
    o = jnp.dot(h, w2_ref[...], preferred_element_type=_F32) + b2_ref[...]
    o_ref[...] = jnp.tanh(o) * PI


def _node_mlp(x, w1, b1, w2, b2):
    blk = 2000
    grid = _N_NODES // blk
    return pl.pallas_call(
        _node_mlp_body,
        grid=(grid,),
        in_specs=[
            pl.BlockSpec((blk, 128), lambda i: (i, 0)),
            pl.BlockSpec((128, 128), lambda i: (0, 0)),
            pl.BlockSpec((1, 128), lambda i: (0, 0)),
            pl.BlockSpec((128, 2), lambda i: (0, 0)),
            pl.BlockSpec((1, 2), lambda i: (0, 0)),
        ],
        out_specs=pl.BlockSpec((blk, 2), lambda i: (i, 0)),
        out_shape=jax.ShapeDtypeStruct((_N_NODES, 2), _F32),
    )(x, w1, b1, w2, b2)


def _edge_mlp(x, w1, b1, w2, b2):
    blk = 8000
    grid = _N_EDGES // blk
    return pl.pallas_call(
        _node_mlp_body,
        grid=(grid,),
        in_specs=[
            pl.BlockSpec((blk, 16), lambda i: (i, 0)),
            pl.BlockSpec((16, 128), lambda i: (0, 0)),
            pl.BlockSpec((1, 128), lambda i: (0, 0)),
            pl.BlockSpec((128, 2), lambda i: (0, 0)),
            pl.BlockSpec((1, 2), lambda i: (0, 0)),
        ],
        out_specs=pl.BlockSpec((blk, 2), lambda i: (i, 0)),
        out_shape=jax.ShapeDtypeStruct((_N_EDGES, 2), _F32),
    )(x, w1, b1, w2, b2)


_CH = 8000
_NCH = _N_OCC // _CH


def _sc_build_tables(occ_node, occ_nbr, zeros_tab):
    mesh = plsc.VectorSubcoreMesh(core_axis_name="c", subcore_axis_name="s")

    @functools.partial(
        pl.kernel,
        out_type=(
            jax.ShapeDtypeStruct((_N_PAD * _K,), _I32),
            jax.ShapeDtypeStruct((_N_PAD * _K,), _I32),
            jax.ShapeDtypeStruct((_N_PAD,), _I32),
        ),
        mesh=mesh,
        compiler_params=_SC_PARAMS,
        scratch_types=[
            pltpu.VMEM((_CH,), _I32),
            pltpu.VMEM((_CH,), _I32),
            pltpu.VMEM((_N_PAD * _K,), _I32),
            pltpu.VMEM((_N_PAD * _K,), _I32),
            pltpu.VMEM((_N_SENT,), _I32),
        ],
    )
    def k(occn_hbm, occb_hbm, z_hbm, nbrn_o, nbre_o, cnt_o, bn, bb, tn, te, tcnt):
        wid = lax.axis_index("s") * 2 + lax.axis_index("c")

        @pl.when(wid == 0)
        def _():
            pltpu.sync_copy(z_hbm, tn)
            pltpu.sync_copy(z_hbm, te)
            pltpu.sync_copy(z_hbm.at[pl.ds(0, _N_SENT)], tcnt)
            iota = lax.iota(_I32, 16)
            ones = jnp.ones((16,), _I32)

            def chunk(kk, carry):
                pltpu.sync_copy(occn_hbm.at[pl.ds(kk * _CH, _CH)], bn)
                pltpu.sync_copy(occb_hbm.at[pl.ds(kk * _CH, _CH)], bb)

                def step(i, c2):
                    node = bn[pl.ds(i * 16, 16)]
                    nbr = bb[pl.ds(i * 16, 16)]
                    pos = kk * _CH + i * 16 + iota
                    eid = lax.shift_right_logical(pos, 1)
                    valid = node != nbr
                    node_eff = jnp.where(valid, node, _N_PAD + iota)
                    cl, _m = plsc.scan_count(node_eff)
                    base = plsc.load_gather(tcnt, [node_eff])
                    rank = base + cl - 1
                    ok = valid & (rank < _K)
                    addr = node * _K + jnp.clip(rank, 0, _K - 1)
                    plsc.store_scatter(tn, [addr], nbr, mask=ok)
                    plsc.store_scatter(te, [addr], eid, mask=ok)
                    plsc.addupdate_scatter(tcnt, [node_eff], ones)
                    return c2

                return lax.fori_loop(0, _CH // 16, step, carry)

            lax.fori_loop(0, _NCH, chunk, 0)
            pltpu.sync_copy(tn, nbrn_o)
            pltpu.sync_copy(te, nbre_o)
            pltpu.sync_copy(tcnt.at[pl.ds(0, _N_PAD)], cnt_o)

    return k(occ_node, occ_nbr, zeros_tab)


_NPW = _N_PAD // 32
_EW = 80


def _sc_gather(nf_flat, ef0, ef1, tn, te):
    mesh = plsc.VectorSubcoreMesh(core_axis_name="c", subcore_axis_name="s")

    @functools.partial(
        pl.kernel,
        out_type=(
            jax.ShapeDtypeStruct((_N_PAD * 8,), _F32),
            jax.ShapeDtypeStruct((_N_PAD * 6,), _F32),
        ),
        mesh=mesh,
        compiler_params=_SC_PARAMS,
        scratch_types=[
            pltpu.VMEM((_N_PAD * 2,), _F32),
            pltpu.VMEM((_NPW * _K,), _I32),
            pltpu.VMEM((_NPW * _K,), _I32),
            pltpu.VMEM((_NPW * 8,), _F32),
            pltpu.VMEM((_NPW * 6,), _F32),
            pltpu.VMEM((_EW,), _F32),
            pltpu.VMEM((_EW,), _F32),
            pltpu.SemaphoreType.DMA,
        ],
    )
    def k(nf_hbm, ef0_hbm, ef1_hbm, tn_hbm, te_hbm, nfeat_o, efeat_o,
          nft, tn_l, te_l, nfb, efb, rows0, rows1, sem):
        wid = lax.axis_index("s") * 2 + lax.axis_index("c")
        base = wid * _NPW
        pltpu.sync_copy(nf_hbm, nft)
        pltpu.sync_copy(tn_hbm.at[pl.ds(base * _K, _NPW * _K)], tn_l)
        pltpu.sync_copy(te_hbm.at[pl.ds(base * _K, _NPW * _K)], te_l)
        iota = lax.iota(_I32, 16)

        def nstep(i, carry):
            l = i * 16 + iota
            gid2 = (base + l) * 2
            for c in (0, 1):
                v = plsc.load_gather(nft, [gid2 + c])
                plsc.store_scatter(nfb, [l * 8 + c], v)
            for s in range(_K):
                ids2 = plsc.load_gather(tn_l, [l * _K + s]) * 2
                for c in (0, 1):
                    v = plsc.load_gather(nft, [ids2 + c])
                    plsc.store_scatter(nfb, [l * 8 + (s + 1) * 2 + c], v)
            return carry

        lax.fori_loop(0, _NPW // 16, nstep, 0)

        def estep(w, carry):
            idx = te_l.at[pl.ds(w * _EW, _EW)]
            cp0 = pltpu.async_copy(ef0_hbm.at[idx], rows0, sem)
            cp1 = pltpu.async_copy(ef1_hbm.at[idx], rows1, sem)
            cp0.wait()
            cp1.wait()
            for j in range(_EW // 16):
                p = w * _EW + j * 16 + iota
                node = p // _K
                slot = p - node * _K
                v0 = rows0[pl.ds(j * 16, 16)]
                v1 = rows1[pl.ds(j * 16, 16)]
                addr = node * 6 + slot * 2
                plsc.store_scatter(efb, [addr], v0)
                plsc.store_scatter(efb, [addr + 1], v1)
            return carry

        lax.fori_loop(0, _NPW * _K // _EW, estep, 0)
        pltpu.sync_copy(nfb, nfeat_o.at[pl.ds(base * 8, _NPW * 8)])
        pltpu.sync_copy(efb, efeat_o.at[pl.ds(base * 6, _NPW * 6)])

    return k(nf_flat, ef0, ef1, tn, te)


def _finish_body(nfeat_ref, efeat_ref, nf_ref, cnt_ref, batch_ref,
                 wqe_ref, wqn_ref, wu1a_ref, wu1b_ref, bu1_ref, wu2_ref,
                 bu2_ref, wh1_ref, bh1_ref, wh2_ref, bh2_ref,
                 o_ref, sums_ref, counts_ref, *, nblk):
    pid = pl.program_id(0)
    nf = nf_ref[...]
    msg = jnp.tanh(
        jnp.dot(efeat_ref[...], wqe_ref[...], preferred_element_type=_F32)
        + jnp.dot(nfeat_ref[...], wqn_ref[...], preferred_element_type=_F32))
    h = _leaky(jnp.dot(nf, wu1a_ref[...], preferred_element_type=_F32)
               + msg * wu1b_ref[...] + bu1_ref[...])
    upd = jnp.dot(h, wu2_ref[...], preferred_element_type=_F32) + bu2_ref[...]
    center = cnt_ref[...] >= _K
    nf2 = jnp.where(center, upd, 0.0) + nf
    onehot = (batch_ref[...] ==
              lax.broadcasted_iota(_I32, (nf.shape[0], _NG), 1)).astype(_F32)
    s_blk = lax.dot_general(onehot, nf2, (((0,), (0,)), ((), ())),
                            preferred_element_type=_F32)
    c_blk = lax.dot_general(onehot, jnp.ones((nf.shape[0], 1), _F32),
                            (((0,), (0,)), ((), ())),
                            preferred_element_type=_F32)

    @pl.when(pid == 0)
    def _():
        sums_ref[...] = s_blk
        counts_ref[...] = c_blk

    @pl.when(pid > 0)
    def _():
        sums_ref[...] += s_blk
        counts_ref[...] += c_blk

    @pl.when(pid == nblk - 1)
    def _():
        ge = sums_ref[...] / jnp.clip(counts_ref[...], 1.0)
        hh = _leaky(jnp.dot(ge, wh1_ref[...], preferred_element_type=_F32)
                    + bh1_ref[...])
        o_ref[...] = jnp.dot(hh, wh2_ref[...],
                             preferred_element_type=_F32) + bh2_ref[...]


def _finish(nfeat, efeat, nf, cnt2d, batch2d,
            wqe, wqn, wu1a, wu1b, bu1, wu2, bu2, wh1, bh1, wh2, bh2):
    blk = 2000
    nblk = _N_NODES // blk
    fixed = lambda r, c: (lambda i: (0, 0))
    return pl.pallas_call(
        functools.partial(_finish_body, nblk=nblk),
        grid=(nblk,),
        in_specs=[
            pl.BlockSpec((blk, 8), lambda i: (i, 0)),
            pl.BlockSpec((blk, 6), lambda i: (i, 0)),
            pl.BlockSpec((blk, 2), lambda i: (i, 0)),
            pl.BlockSpec((blk, 1), lambda i: (i, 0)),
            pl.BlockSpec((blk, 1), lambda i: (i, 0)),
            pl.BlockSpec((6, 1), fixed(6, 1)),
            pl.BlockSpec((8, 1), fixed(8, 1)),
            pl.BlockSpec((2, 128), fixed(2, 128)),
            pl.BlockSpec((1, 128), fixed(1, 128)),
            pl.BlockSpec((1, 128), fixed(1, 128)),
            pl.BlockSpec((128, 2), fixed(128, 2)),
            pl.BlockSpec((1, 2), fixed(1, 2)),
            pl.BlockSpec((2, 2), fixed(2, 2)),
            pl.BlockSpec((1, 2), fixed(1, 2)),
            pl.BlockSpec((2, 2), fixed(2, 2)),
            pl.BlockSpec((1, 2), fixed(1, 2)),
        ],
        out_specs=pl.BlockSpec((_NG, 2), lambda i: (0, 0)),
        out_shape=jax.ShapeDtypeStruct((_NG, 2), _F32),
        scratch_shapes=[
            pltpu.VMEM((_NG, 2), _F32),
            pltpu.VMEM((_NG, 1), _F32),
        ],
    )(nfeat, efeat, nf, cnt2d, batch2d,
      wqe, wqn, wu1a, wu1b, bu1, wu2, bu2, wh1, bh1, wh2, bh2)


def kernel(node_feat, edge_attr, edge_index, batch,
           Wn1, bn1, Wn2, bn2, We1, be1, We2, be2, wq,
           Wu1, bu1, Wu2, bu2, Wh1, bh1, Wh2, bh2):
    src = edge_index[0].astype(_I32)
    dst = edge_index[1].astype(_I32)
    occ_node = jnp.stack([src, dst], axis=1).reshape(-1)
    occ_nbr = jnp.stack([dst, src], axis=1).reshape(-1)
    zeros_tab = jnp.zeros((_N_PAD * _K,), _I32)

    nf = _node_mlp(node_feat, Wn1, bn1.reshape(1, -1), Wn2, bn2.reshape(1, -1))
    ef = _edge_mlp(edge_attr, We1, be1.reshape(1, -1), We2, be2.reshape(1, -1))
    tn, te, cnt = _sc_build_tables(occ_node, occ_nbr, zeros_tab)

    nf_flat = jnp.zeros((_N_PAD, 2), _F32).at[:_N_NODES].set(nf).reshape(-1)
    nfeat_flat, efeat_flat = _sc_gather(nf_flat, ef[:, 0], ef[:, 1], tn, te)
    nfeat = nfeat_flat.reshape(_N_PAD, 8)[:_N_NODES]
    efeat = efeat_flat.reshape(_N_PAD, 6)[:_N_NODES]

    return _finish(
        nfeat, efeat, nf,
        cnt[:_N_NODES].reshape(_N_NODES, 1),
        batch.astype(_I32).reshape(_N_NODES, 1),
        wq[:6], wq[6:],
        Wu1[:2], Wu1[2:3], bu1.reshape(1, -1), Wu2, bu2.reshape(1, -1),
        Wh1, bh1.reshape(1, -1), Wh2, bh2.reshape(1, -1))
